# baseline (speedup 1.0000x reference)
import functools

import jax
import jax.numpy as jnp
from jax import lax
from jax.experimental import pallas as pl
from jax.experimental.pallas import tpu as pltpu
from jax.experimental.pallas import tpu_sc as plsc

E = 64
D = 2048
B = 4
NW = 32
KPW = D // NW
CCH = 4
CW = D // CCH
GW = 128
NG = CW // GW


def _ln_kernel(x_ref, gamma_ref, beta_ref, out_ref):
    x = x_ref[:, 7, :]
    mu = jnp.mean(x, axis=-1, keepdims=True)
    var = jnp.mean((x - mu) ** 2, axis=-1, keepdims=True)
    out_ref[...] = ((x - mu) * jax.lax.rsqrt(var + 1e-5)
                    * gamma_ref[...] + beta_ref[...])


def _m_kernel(w2_ref, emb_ref, m_ref):
    m_ref[...] = jax.lax.dot_general(
        w2_ref[...], emb_ref[...], (((1,), (1,)), ((), ())),
        preferred_element_type=jnp.float32)


def _sc_mlp1(xsplat_hbm, w1_hbm, out_hbm, xb_v, w_v0, w_v1, out_v,
             sem0, sem1):
    wid = lax.axis_index("c") * 16 + lax.axis_index("s")
    row0 = wid * KPW
    pltpu.sync_copy(xsplat_hbm.at[wid], xb_v)

    bufs = (w_v0, w_v1)
    sems = (sem0, sem1)
    copies = [None] * CCH
    copies[0] = pltpu.async_copy(
        w1_hbm.at[pl.ds(row0, KPW), pl.ds(0, CW)], bufs[0], sems[0])
    for ch in range(CCH):
        if ch + 1 < CCH:
            copies[ch + 1] = pltpu.async_copy(
                w1_hbm.at[pl.ds(row0, KPW), pl.ds((ch + 1) * CW, CW)],
                bufs[(ch + 1) % 2], sems[(ch + 1) % 2])
        copies[ch].wait()
        w_ref = bufs[ch % 2]
        for g in range(NG):
            def body(k4, accs):
                accs = list(accs)
                for dk in range(4):
                    k = k4 * 4 + dk
                    xbs = [xb_v[k, b, :] for b in range(B)]
                    for i in range(GW // 16):
                        w = w_ref[k, pl.ds(g * GW + i * 16, 16)]
                        for b in range(B):
                            accs[b * (GW // 16) + i] = (
                                accs[b * (GW // 16) + i] + w * xbs[b])
                return tuple(accs)

            init = tuple(jnp.zeros((16,), jnp.float32)
                         for _ in range(B * (GW // 16)))
            accs = lax.fori_loop(0, KPW // 4, body, init)
            for i in range(GW // 16):
                for b in range(B):
                    out_v[b, pl.ds(ch * CW + g * GW + i * 16, 16)] = (
                        accs[b * (GW // 16) + i])
    pltpu.sync_copy(out_v, out_hbm.at[wid])


def _final_kernel(idx_ref,
                  part_ref,
                  b1_ref, b2_ref,
                  m_ref,
                  emb_ref,
                  cent_ref,
                  spat_ref, comp_ref,
                  out_ref):
    h1 = jnp.sum(part_ref[...], axis=0) + b1_ref[...]
    h1 = 0.5 * h1 * (1.0 + jax.lax.erf(h1 * 0.7071067811865476))
    att = jnp.dot(h1, m_ref[...], preferred_element_type=jnp.float32)
    r = jax.lax.dot_general(
        b2_ref[...], emb_ref[...], (((1,), (1,)), ((), ())),
        preferred_element_type=jnp.float32)
    att = (att + r) * (1.0 / (D ** 0.5))
    cent = jax.nn.softmax(cent_ref[...], axis=-1)
    idx = idx_ref[0, 0]
    row = spat_ref[pl.ds(idx, 1), :] + comp_ref[pl.ds(idx, 1), :]
    eids = jax.lax.broadcasted_iota(jnp.int32, (1, E), 1)
    row = row + jnp.where(eids == idx, -0.1, 0.0)
    out_ref[...] = jax.nn.softmax(att + cent + row, axis=-1)


def kernel(hidden_states, ln_gamma, ln_beta, W1, b1, W2, b2,
           expert_embeddings, centrality_bias, spatial_bias,
           compatibility_matrix, current_expert_idx):
    _, S, d = hidden_states.shape

    xln = pl.pallas_call(
        _ln_kernel,
        grid=(1,),
        in_specs=[
            pl.BlockSpec((B, 8, d), lambda i: (0, S // 8 - 1, 0)),
            pl.BlockSpec((1, d), lambda i: (0, 0)),
            pl.BlockSpec((1, d), lambda i: (0, 0)),
        ],
        out_specs=pl.BlockSpec((B, d), lambda i: (0, 0)),
        out_shape=jax.ShapeDtypeStruct((B, d), jnp.float32),
    )(hidden_states, ln_gamma.reshape(1, d), ln_beta.reshape(1, d))

    xsplat = jnp.broadcast_to(
        xln.T.reshape(NW, KPW, B, 1), (NW, KPW, B, 16))

    mesh = plsc.VectorSubcoreMesh(core_axis_name="c", subcore_axis_name="s")
    sc_call = functools.partial(
        pl.kernel, _sc_mlp1, mesh=mesh,
        out_type=jax.ShapeDtypeStruct((NW, B, d), jnp.float32),
        scratch_types=[
            pltpu.VMEM((KPW, B, 16), jnp.float32),
            pltpu.VMEM((KPW, CW), jnp.float32),
            pltpu.VMEM((KPW, CW), jnp.float32),
            pltpu.VMEM((B, d), jnp.float32),
            pltpu.SemaphoreType.DMA,
            pltpu.SemaphoreType.DMA,
        ],
    )()
    partials = sc_call(xsplat, W1)

    M = pl.pallas_call(
        _m_kernel,
        grid=(4,),
        in_specs=[
            pl.BlockSpec((d // 4, d), lambda j: (j, 0)),
            pl.BlockSpec((E, d), lambda j: (0, 0)),
        ],
        out_specs=pl.BlockSpec((d // 4, E), lambda j: (j, 0)),
        out_shape=jax.ShapeDtypeStruct((d, E), jnp.float32),
        compiler_params=pltpu.CompilerParams(
            dimension_semantics=("arbitrary",),
        ),
    )(W2, expert_embeddings)

    idx = jnp.asarray(current_expert_idx, jnp.int32).reshape(1, 1)
    grid_spec = pltpu.PrefetchScalarGridSpec(
        num_scalar_prefetch=1,
        grid=(1,),
        in_specs=[
            pl.BlockSpec((NW, B, d), lambda j, *_: (0, 0, 0)),
            pl.BlockSpec((1, d), lambda j, *_: (0, 0)),
            pl.BlockSpec((1, d), lambda j, *_: (0, 0)),
            pl.BlockSpec((d, E), lambda j, *_: (0, 0)),
            pl.BlockSpec((E, d), lambda j, *_: (0, 0)),
            pl.BlockSpec((1, E), lambda j, *_: (0, 0)),
            pl.BlockSpec((E, E), lambda j, *_: (0, 0)),
            pl.BlockSpec((E, E), lambda j, *_: (0, 0)),
        ],
        out_specs=pl.BlockSpec((B, E), lambda j, *_: (0, 0)),
    )
    return pl.pallas_call(
        _final_kernel,
        grid_spec=grid_spec,
        out_shape=jax.ShapeDtypeStruct((B, E), jnp.float32),
    )(idx, partials, b1.reshape(1, d), b2.reshape(1, d), M,
      expert_embeddings, centrality_bias.reshape(1, E),
      spatial_bias, compatibility_matrix)


# speedup vs baseline: 1.4486x; 1.4486x over previous
import functools

import jax
import jax.numpy as jnp
from jax import lax
from jax.experimental import pallas as pl
from jax.experimental.pallas import tpu as pltpu
from jax.experimental.pallas import tpu_sc as plsc

E = 64
D = 2048
B = 4
NW = 32
KPW = D // NW
CCH = 4
CW = D // CCH
GW = 128
NG = CW // GW


def _ln_kernel(x_ref, gamma_ref, beta_ref, out_ref):
    x = x_ref[:, 7, :]
    mu = jnp.mean(x, axis=-1, keepdims=True)
    var = jnp.mean((x - mu) ** 2, axis=-1, keepdims=True)
    out_ref[...] = ((x - mu) * jax.lax.rsqrt(var + 1e-5)
                    * gamma_ref[...] + beta_ref[...])


def _m_kernel(w2_ref, emb_ref, m_ref):
    m_ref[...] = jax.lax.dot_general(
        w2_ref[...], emb_ref[...], (((1,), (1,)), ((), ())),
        preferred_element_type=jnp.float32)


def _sc_mlp1(xsplat_hbm, w1_hbm, out_hbm, xb_v, w_v0, w_v1, out_v,
             sem0, sem1):
    wid = lax.axis_index("c") * 16 + lax.axis_index("s")
    row0 = wid * KPW
    pltpu.sync_copy(xsplat_hbm.at[wid], xb_v)

    bufs = (w_v0, w_v1)
    sems = (sem0, sem1)
    copies = [None] * CCH
    copies[0] = pltpu.async_copy(
        w1_hbm.at[pl.ds(row0, KPW), pl.ds(0, CW)], bufs[0], sems[0])
    for ch in range(CCH):
        if ch + 1 < CCH:
            copies[ch + 1] = pltpu.async_copy(
                w1_hbm.at[pl.ds(row0, KPW), pl.ds((ch + 1) * CW, CW)],
                bufs[(ch + 1) % 2], sems[(ch + 1) % 2])
        copies[ch].wait()
        w_ref = bufs[ch % 2]
        for g in range(NG):
            init = tuple(jnp.zeros((16,), jnp.float32)
                         for _ in range(B * (GW // 16)))

            @plsc.parallel_loop(0, KPW, unroll=2, carry=init)
            def accs(k, accs):
                accs = list(accs)
                xbs = [xb_v[k, b, :] for b in range(B)]
                for i in range(GW // 16):
                    w = w_ref[k, pl.ds(g * GW + i * 16, 16)]
                    for b in range(B):
                        accs[b * (GW // 16) + i] = (
                            accs[b * (GW // 16) + i] + w * xbs[b])
                return tuple(accs)
            for i in range(GW // 16):
                for b in range(B):
                    out_v[b, pl.ds(ch * CW + g * GW + i * 16, 16)] = (
                        accs[b * (GW // 16) + i])
    pltpu.sync_copy(out_v, out_hbm.at[wid])


def _final_kernel(idx_ref,
                  part_ref,
                  b1_ref, b2_ref,
                  m_ref,
                  emb_ref,
                  cent_ref,
                  spat_ref, comp_ref,
                  out_ref):
    h1 = jnp.sum(part_ref[...], axis=0) + b1_ref[...]
    h1 = 0.5 * h1 * (1.0 + jax.lax.erf(h1 * 0.7071067811865476))
    att = jnp.dot(h1, m_ref[...], preferred_element_type=jnp.float32)
    r = jax.lax.dot_general(
        b2_ref[...], emb_ref[...], (((1,), (1,)), ((), ())),
        preferred_element_type=jnp.float32)
    att = (att + r) * (1.0 / (D ** 0.5))
    cent = jax.nn.softmax(cent_ref[...], axis=-1)
    idx = idx_ref[0, 0]
    row = spat_ref[pl.ds(idx, 1), :] + comp_ref[pl.ds(idx, 1), :]
    eids = jax.lax.broadcasted_iota(jnp.int32, (1, E), 1)
    row = row + jnp.where(eids == idx, -0.1, 0.0)
    out_ref[...] = jax.nn.softmax(att + cent + row, axis=-1)


def kernel(hidden_states, ln_gamma, ln_beta, W1, b1, W2, b2,
           expert_embeddings, centrality_bias, spatial_bias,
           compatibility_matrix, current_expert_idx):
    _, S, d = hidden_states.shape

    xln = pl.pallas_call(
        _ln_kernel,
        grid=(1,),
        in_specs=[
            pl.BlockSpec((B, 8, d), lambda i: (0, S // 8 - 1, 0)),
            pl.BlockSpec((1, d), lambda i: (0, 0)),
            pl.BlockSpec((1, d), lambda i: (0, 0)),
        ],
        out_specs=pl.BlockSpec((B, d), lambda i: (0, 0)),
        out_shape=jax.ShapeDtypeStruct((B, d), jnp.float32),
    )(hidden_states, ln_gamma.reshape(1, d), ln_beta.reshape(1, d))

    xsplat = jnp.broadcast_to(
        xln.T.reshape(NW, KPW, B, 1), (NW, KPW, B, 16))

    mesh = plsc.VectorSubcoreMesh(core_axis_name="c", subcore_axis_name="s")
    sc_call = functools.partial(
        pl.kernel, _sc_mlp1, mesh=mesh,
        out_type=jax.ShapeDtypeStruct((NW, B, d), jnp.float32),
        scratch_types=[
            pltpu.VMEM((KPW, B, 16), jnp.float32),
            pltpu.VMEM((KPW, CW), jnp.float32),
            pltpu.VMEM((KPW, CW), jnp.float32),
            pltpu.VMEM((B, d), jnp.float32),
            pltpu.SemaphoreType.DMA,
            pltpu.SemaphoreType.DMA,
        ],
    )()
    partials = sc_call(xsplat, W1)

    M = pl.pallas_call(
        _m_kernel,
        grid=(4,),
        in_specs=[
            pl.BlockSpec((d // 4, d), lambda j: (j, 0)),
            pl.BlockSpec((E, d), lambda j: (0, 0)),
        ],
        out_specs=pl.BlockSpec((d // 4, E), lambda j: (j, 0)),
        out_shape=jax.ShapeDtypeStruct((d, E), jnp.float32),
        compiler_params=pltpu.CompilerParams(
            dimension_semantics=("arbitrary",),
        ),
    )(W2, expert_embeddings)

    idx = jnp.asarray(current_expert_idx, jnp.int32).reshape(1, 1)
    grid_spec = pltpu.PrefetchScalarGridSpec(
        num_scalar_prefetch=1,
        grid=(1,),
        in_specs=[
            pl.BlockSpec((NW, B, d), lambda j, *_: (0, 0, 0)),
            pl.BlockSpec((1, d), lambda j, *_: (0, 0)),
            pl.BlockSpec((1, d), lambda j, *_: (0, 0)),
            pl.BlockSpec((d, E), lambda j, *_: (0, 0)),
            pl.BlockSpec((E, d), lambda j, *_: (0, 0)),
            pl.BlockSpec((1, E), lambda j, *_: (0, 0)),
            pl.BlockSpec((E, E), lambda j, *_: (0, 0)),
            pl.BlockSpec((E, E), lambda j, *_: (0, 0)),
        ],
        out_specs=pl.BlockSpec((B, E), lambda j, *_: (0, 0)),
    )
    return pl.pallas_call(
        _final_kernel,
        grid_spec=grid_spec,
        out_shape=jax.ShapeDtypeStruct((B, E), jnp.float32),
    )(idx, partials, b1.reshape(1, d), b2.reshape(1, d), M,
      expert_embeddings, centrality_bias.reshape(1, E),
      spatial_bias, compatibility_matrix)


# speedup vs baseline: 5.2748x; 3.6414x over previous
import jax
import jax.numpy as jnp
from jax.experimental import pallas as pl
from jax.experimental.pallas import tpu as pltpu

E = 64
D = 2048
TILE = 512
NSTEPS = 2


def _router_kernel(idx_ref,
                   x_ref,
                   gamma_ref, beta_ref,
                   w1a_ref, w1b_ref,
                   b1a_ref, b1b_ref,
                   w2a_ref, w2b_ref,
                   b2_ref,
                   emb_ref,
                   cent_ref,
                   spat_ref,
                   comp_ref,
                   out_ref,
                   xln_ref,
                   acc_ref):
    j = pl.program_id(0)

    @pl.when(j == 0)
    def _init():
        x = x_ref[:, 7, :]
        mu = jnp.mean(x, axis=-1, keepdims=True)
        var = jnp.mean((x - mu) ** 2, axis=-1, keepdims=True)
        xln_ref[...] = ((x - mu) * jax.lax.rsqrt(var + 1e-5)
                        * gamma_ref[...] + beta_ref[...])
        acc_ref[...] = jnp.zeros_like(acc_ref)

    xln = xln_ref[...]
    c = 0.7071067811865476
    h1a = jnp.dot(xln, w1a_ref[...],
                  preferred_element_type=jnp.float32) + b1a_ref[...]
    h1a = 0.5 * h1a * (1.0 + jax.lax.erf(h1a * c))
    h1b = jnp.dot(xln, w1b_ref[...],
                  preferred_element_type=jnp.float32) + b1b_ref[...]
    h1b = 0.5 * h1b * (1.0 + jax.lax.erf(h1b * c))
    acc_ref[...] += (jnp.dot(h1a, w2a_ref[...],
                             preferred_element_type=jnp.float32)
                     + jnp.dot(h1b, w2b_ref[...],
                               preferred_element_type=jnp.float32))

    @pl.when(j == NSTEPS - 1)
    def _finish():
        h2 = acc_ref[...] + b2_ref[...]
        att = jax.lax.dot_general(
            h2, emb_ref[...], (((1,), (1,)), ((), ())),
            preferred_element_type=jnp.float32) * (1.0 / (D ** 0.5))
        cent = jax.nn.softmax(cent_ref[...], axis=-1)
        idx = idx_ref[0, 0]
        row = spat_ref[pl.ds(idx, 1), :] + comp_ref[pl.ds(idx, 1), :]
        eids = jax.lax.broadcasted_iota(jnp.int32, (1, E), 1)
        row = row + jnp.where(eids == idx, -0.1, 0.0)
        out_ref[...] = jax.nn.softmax(att + cent + row, axis=-1)


def kernel(hidden_states, ln_gamma, ln_beta, W1, b1, W2, b2,
           expert_embeddings, centrality_bias, spatial_bias,
           compatibility_matrix, current_expert_idx):
    B, S, d = hidden_states.shape
    idx = jnp.asarray(current_expert_idx, jnp.int32).reshape(1, 1)
    grid_spec = pltpu.PrefetchScalarGridSpec(
        num_scalar_prefetch=1,
        grid=(NSTEPS,),
        in_specs=[
            pl.BlockSpec((B, 8, d), lambda j, *_: (0, S // 8 - 1, 0)),
            pl.BlockSpec((1, d), lambda j, *_: (0, 0)),
            pl.BlockSpec((1, d), lambda j, *_: (0, 0)),
            pl.BlockSpec((d, TILE), lambda j, *_: (0, j)),
            pl.BlockSpec((d, TILE), lambda j, *_: (0, j + NSTEPS)),
            pl.BlockSpec((1, TILE), lambda j, *_: (0, j)),
            pl.BlockSpec((1, TILE), lambda j, *_: (0, j + NSTEPS)),
            pl.BlockSpec((TILE, d), lambda j, *_: (j, 0)),
            pl.BlockSpec((TILE, d), lambda j, *_: (j + NSTEPS, 0)),
            pl.BlockSpec((1, d), lambda j, *_: (0, 0)),
            pl.BlockSpec((E, d), lambda j, *_: (0, 0)),
            pl.BlockSpec((1, E), lambda j, *_: (0, 0)),
            pl.BlockSpec((E, E), lambda j, *_: (0, 0)),
            pl.BlockSpec((E, E), lambda j, *_: (0, 0)),
        ],
        out_specs=pl.BlockSpec((B, E), lambda j, *_: (0, 0)),
        scratch_shapes=[
            pltpu.VMEM((B, d), jnp.float32),
            pltpu.VMEM((B, d), jnp.float32),
        ],
    )
    return pl.pallas_call(
        _router_kernel,
        grid_spec=grid_spec,
        out_shape=jax.ShapeDtypeStruct((B, E), jnp.float32),
        compiler_params=pltpu.CompilerParams(
            dimension_semantics=("arbitrary",),
        ),
    )(idx,
      hidden_states,
      ln_gamma.reshape(1, d), ln_beta.reshape(1, d),
      W1, W1, b1.reshape(1, d), b1.reshape(1, d),
      W2, W2, b2.reshape(1, d),
      expert_embeddings,
      centrality_bias.reshape(1, E),
      spatial_bias, compatibility_matrix)
